# baseline (speedup 1.0000x reference)
import functools

import jax
import jax.numpy as jnp
from jax import lax
from jax.experimental import pallas as pl
from jax.experimental.pallas import tpu as pltpu
from jax.experimental.pallas import tpu_sc as plsc

VOCAB = 1000000
B = 4096
L = 200
EMB = 32

NC = 2
NS = 16
NW = NC * NS

CW = 128
NFULL = VOCAB // CW
TAILW = VOCAB - NFULL * CW
NCHUNKS = NFULL + 1
CHUNK_ELS = CW * EMB
EXT = NCHUNKS * CHUNK_ELS
NPAIR_A = (NFULL // NW + 2) // 2 + 1

BB = B // NW
K = 4
NBATCH = L // K
NPAIR_G = NBATCH // 2


def _sc_relayout(table_t):
    mesh = plsc.VectorSubcoreMesh(
        core_axis_name="c", subcore_axis_name="s",
        num_cores=NC, num_subcores=NS)

    @functools.partial(
        pl.kernel,
        mesh=mesh,
        out_type=jax.ShapeDtypeStruct((EXT,), jnp.float32),
        scratch_types=[
            pltpu.VMEM((2, EMB, CW), jnp.float32),
            pltpu.VMEM((2, CHUNK_ELS), jnp.float32),
            pltpu.SemaphoreType.DMA,
            pltpu.SemaphoreType.DMA,
        ],
        compiler_params=pltpu.CompilerParams(
            use_tc_tiling_on_sc=False, needs_layout_passes=False),
    )
    def k(tab_hbm, out_hbm, in_v, out_v, rsem, wsem):
        wid = lax.axis_index("s") * NC + lax.axis_index("c")
        nt = 244 + (wid <= 4).astype(jnp.int32)

        iota = lax.broadcasted_iota(jnp.int32, (16,), 0)
        flat_bases = [(16 * xg + iota) * EMB for xg in range(CW // 16)]

        def fire_read(t, p):
            ci = wid + NW * t
            pltpu.async_copy(
                tab_hbm.at[:, pl.ds(ci * CW, CW)], in_v.at[p], rsem)

        def wait_read(p):
            pltpu.make_async_copy(
                tab_hbm.at[:, pl.ds(0, CW)], in_v.at[p], rsem).wait()

        def transpose(p, nxg):
            for c in range(EMB):
                for xg in range(nxg):
                    val = in_v[p, c, pl.ds(16 * xg, 16)]
                    plsc.store_scatter(
                        out_v.at[p], [flat_bases[xg] + c], val)

        def fire_write(t, p):
            ci = wid + NW * t
            pltpu.async_copy(
                out_v.at[p],
                out_hbm.at[pl.ds(ci * CHUNK_ELS, CHUNK_ELS)], wsem)

        def wait_write(p):
            pltpu.make_async_copy(
                out_v.at[p], out_hbm.at[pl.ds(0, CHUNK_ELS)], wsem).wait()

        def step(t, p):
            @pl.when(t < nt)
            def _():
                ci = wid + NW * t

                @pl.when((t + 1 < nt) & (ci + NW < NFULL))
                def _():
                    fire_read(t + 1, 1 - p)

                @pl.when(t >= 2)
                def _():
                    wait_write(p)

                @pl.when(ci < NFULL)
                def _():
                    wait_read(p)
                    transpose(p, CW // 16)

                @pl.when(ci == NFULL)
                def _():
                    pltpu.sync_copy(
                        tab_hbm.at[:, pl.ds(NFULL * CW, TAILW)],
                        in_v.at[p].at[:, pl.ds(0, TAILW)])
                    transpose(p, TAILW // 16)

                fire_write(t, p)

        fire_read(0, 0)

        def body(u, carry):
            step(2 * u, 0)
            step(2 * u + 1, 1)
            return carry

        lax.fori_loop(0, NPAIR_A, body, 0)
        wait_write(0)
        wait_write(1)

    return k(table_t)


def _sc_gather(idx_lb, table_rm):
    mesh = plsc.VectorSubcoreMesh(
        core_axis_name="c", subcore_axis_name="s",
        num_cores=NC, num_subcores=NS)

    @functools.partial(
        pl.kernel,
        mesh=mesh,
        out_type=jax.ShapeDtypeStruct((L, B, EMB), jnp.float32),
        scratch_types=[
            pltpu.VMEM((L, BB), jnp.int32),
            pltpu.VMEM((2, K, BB, EMB), jnp.float32),
            pltpu.SemaphoreType.DMA,
            pltpu.SemaphoreType.DMA,
        ],
        compiler_params=pltpu.CompilerParams(use_tc_tiling_on_sc=False),
    )
    def k(idx_hbm, table_hbm, out_hbm, idx_v, rows_v, gsem, wsem):
        wid = lax.axis_index("s") * NC + lax.axis_index("c")
        b0 = wid * BB
        pltpu.sync_copy(idx_hbm.at[:, pl.ds(b0, BB)], idx_v)

        def fire(t, p):
            return [
                pltpu.async_copy(
                    table_hbm.at[idx_v.at[K * t + j]],
                    rows_v.at[p].at[j],
                    gsem)
                for j in range(K)
            ]

        def writeback(t, p):
            pltpu.async_copy(
                rows_v.at[p],
                out_hbm.at[pl.ds(K * t, K), pl.ds(b0, BB)],
                wsem)

        def wait_writeback(p):
            pltpu.make_async_copy(
                rows_v.at[p],
                out_hbm.at[pl.ds(0, K), pl.ds(b0, BB)],
                wsem).wait()

        def body(t, carry):
            a = 2 * t

            @pl.when(t > 0)
            def _():
                wait_writeback(0)

            ga = fire(a, 0)

            @pl.when(t > 0)
            def _():
                wait_writeback(1)

            gb = fire(a + 1, 1)
            for cp in ga:
                cp.wait()
            writeback(a, 0)
            for cp in gb:
                cp.wait()
            writeback(a + 1, 1)
            return carry

        lax.fori_loop(0, NPAIR_G, body, 0)
        wait_writeback(0)
        wait_writeback(1)

    return k(idx_lb, table_rm)


def kernel(inputs, table):
    table_t = jnp.swapaxes(table, 0, 1)
    flat = _sc_relayout(table_t)
    table_rm = jnp.reshape(flat[:VOCAB * EMB], (VOCAB, EMB))
    idx_lb = jnp.swapaxes(inputs, 0, 1).astype(jnp.int32)
    out = _sc_gather(idx_lb, table_rm)
    return jnp.transpose(out, (1, 0, 2))


# speedup vs baseline: 3.0553x; 3.0553x over previous
import functools

import jax
import jax.numpy as jnp
from jax import lax
from jax.experimental import pallas as pl
from jax.experimental.pallas import tpu as pltpu
from jax.experimental.pallas import tpu_sc as plsc

VOCAB = 1000000
B = 4096
L = 200
EMB = 32

NC = 2
NS = 16
NW = NC * NS

BB = B // NW
CT = EMB // 8


def _sc_gather(idx_lb, table):
    mesh = plsc.VectorSubcoreMesh(
        core_axis_name="c", subcore_axis_name="s",
        num_cores=NC, num_subcores=NS)

    @functools.partial(
        pl.kernel,
        mesh=mesh,
        out_type=jax.ShapeDtypeStruct((L, CT, NW, 8, BB), jnp.float32),
        scratch_types=[
            pltpu.VMEM((L, BB), jnp.int32),
            pltpu.VMEM((2, BB, EMB), jnp.float32),
            pltpu.VMEM((2, CT, 8, BB), jnp.float32),
            pltpu.SemaphoreType.DMA,
            pltpu.SemaphoreType.DMA,
        ],
        compiler_params=pltpu.CompilerParams(
            use_tc_tiling_on_sc=False, needs_layout_passes=False),
    )
    def k(idx_hbm, table_hbm, out_hbm, idx_v, rows_v, tile_v, gsem, wsem):
        wid = lax.axis_index("s") * NC + lax.axis_index("c")
        b0 = wid * BB
        pltpu.sync_copy(idx_hbm.at[:, pl.ds(b0, BB)], idx_v)

        iota = lax.broadcasted_iota(jnp.int32, (16,), 0)
        bl_vecs = [16 * g + iota for g in range(BB // 16)]

        def fire_gather(l, p):
            pltpu.async_copy(
                table_hbm.at[idx_v.at[l]], rows_v.at[p], gsem)

        def wait_gather(p):
            pltpu.make_async_copy(
                table_hbm.at[idx_v.at[0]], rows_v.at[p], gsem).wait()

        def transpose(p):
            for ct in range(CT):
                for cs in range(8):
                    c = jnp.full((16,), 8 * ct + cs, jnp.int32)
                    for g in range(BB // 16):
                        val = plsc.load_gather(
                            rows_v.at[p], [bl_vecs[g], c])
                        tile_v[p, ct, cs, pl.ds(16 * g, 16)] = val

        def fire_write(l, p):
            pltpu.async_copy(
                tile_v.at[p], out_hbm.at[l, :, wid], wsem)

        def wait_write(p):
            pltpu.make_async_copy(
                tile_v.at[p], out_hbm.at[0, :, wid], wsem).wait()

        def step(l, p):
            @pl.when(l + 1 < L)
            def _():
                fire_gather(l + 1, 1 - p)

            wait_gather(p)

            @pl.when(l >= 2)
            def _():
                wait_write(p)

            transpose(p)
            fire_write(l, p)

        fire_gather(0, 0)

        def body(u, carry):
            step(2 * u, 0)
            step(2 * u + 1, 1)
            return carry

        lax.fori_loop(0, L // 2, body, 0)
        wait_write(0)
        wait_write(1)

    return k(idx_lb, table)


def kernel(inputs, table):
    idx_lb = jnp.swapaxes(inputs, 0, 1).astype(jnp.int32)
    out5 = _sc_gather(idx_lb, table)
    out = jnp.transpose(out5, (2, 4, 0, 1, 3)).reshape(B, L, EMB)
    return out


# speedup vs baseline: 3.6081x; 1.1809x over previous
import functools

import jax
import jax.numpy as jnp
from jax import lax
from jax.experimental import pallas as pl
from jax.experimental.pallas import tpu as pltpu
from jax.experimental.pallas import tpu_sc as plsc

VOCAB = 1000000
B = 4096
L = 200
EMB = 32

NC = 2
NS = 16
NW = NC * NS

BB = B // NW
CT = EMB // 8
LSTRIDE = CT * NW * 8 * BB
TILE_ELS = 8 * BB
BLK = CT * TILE_ELS
NLOADS = BB * EMB // 16


def _sc_gather(idx_lb, table):
    mesh = plsc.VectorSubcoreMesh(
        core_axis_name="c", subcore_axis_name="s",
        num_cores=NC, num_subcores=NS)

    @functools.partial(
        pl.kernel,
        mesh=mesh,
        out_type=jax.ShapeDtypeStruct((L * LSTRIDE,), jnp.float32),
        scratch_types=[
            pltpu.VMEM((L, BB), jnp.int32),
            pltpu.VMEM((4, BB, EMB), jnp.float32),
            pltpu.VMEM((2, BLK), jnp.float32),
            pltpu.SemaphoreType.DMA,
            pltpu.SemaphoreType.DMA,
        ],
        compiler_params=pltpu.CompilerParams(
            use_tc_tiling_on_sc=False, needs_layout_passes=False),
    )
    def k(idx_hbm, table_hbm, out_hbm, idx_v, rows_v, tile_v, gsem, wsem):
        wid = lax.axis_index("s") * NC + lax.axis_index("c")
        b0 = wid * BB
        pltpu.sync_copy(idx_hbm.at[:, pl.ds(b0, BB)], idx_v)

        iota = lax.broadcasted_iota(jnp.int32, (16,), 0)
        dst_base = [iota * BB, (iota + 16) * BB]

        def fire_gather(l, p):
            pltpu.async_copy(
                table_hbm.at[idx_v.at[l]], rows_v.at[p], gsem)

        def wait_gather(p):
            pltpu.make_async_copy(
                table_hbm.at[idx_v.at[0]], rows_v.at[p], gsem).wait()

        def transpose(p, q):
            for m in range(NLOADS):
                val = rows_v[p, m // 2, pl.ds(16 * (m % 2), 16)]
                plsc.store_scatter(
                    tile_v.at[q], [dst_base[m % 2] + (m // 2)], val)

        def fire_write(l, q):
            base = l * LSTRIDE + wid * TILE_ELS
            for ct in range(CT):
                pltpu.async_copy(
                    tile_v.at[q].at[pl.ds(ct * TILE_ELS, TILE_ELS)],
                    out_hbm.at[pl.ds(base + ct * NW * TILE_ELS, TILE_ELS)],
                    wsem)

        def wait_write(q):
            for ct in range(CT):
                pltpu.make_async_copy(
                    tile_v.at[q].at[pl.ds(ct * TILE_ELS, TILE_ELS)],
                    out_hbm.at[pl.ds(ct * TILE_ELS, TILE_ELS)],
                    wsem).wait()

        def step(l, p, q):
            @pl.when(l + 3 < L)
            def _():
                fire_gather(l + 3, (p + 3) % 4)

            wait_gather(p)

            @pl.when(l >= 2)
            def _():
                wait_write(q)

            transpose(p, q)
            fire_write(l, q)

        fire_gather(0, 0)
        fire_gather(1, 1)
        fire_gather(2, 2)

        def body(u, carry):
            for r in range(4):
                step(4 * u + r, r, r % 2)
            return carry

        lax.fori_loop(0, L // 4, body, 0)
        wait_write(0)
        wait_write(1)

    return k(idx_lb, table)


def kernel(inputs, table):
    idx_lb = jnp.swapaxes(inputs, 0, 1).astype(jnp.int32)
    flat = _sc_gather(idx_lb, table)
    out5 = flat.reshape(L, CT, NW, 8, BB)
    return jnp.transpose(out5, (2, 4, 0, 1, 3)).reshape(B, L, EMB)


# speedup vs baseline: 3.8332x; 1.0624x over previous
import functools

import jax
import jax.numpy as jnp
from jax import lax
from jax.experimental import pallas as pl
from jax.experimental.pallas import tpu as pltpu
from jax.experimental.pallas import tpu_sc as plsc

VOCAB = 1000000
B = 4096
L = 200
EMB = 32

NC = 2
NS = 16
NW = NC * NS

BB = B // NW
CT = EMB // 8
LSTRIDE = CT * NW * 8 * BB
TILE_ELS = 8 * BB
BLK = CT * TILE_ELS
NLOADS = BB * EMB // 16


def _sc_gather(idx_lb, table):
    mesh = plsc.VectorSubcoreMesh(
        core_axis_name="c", subcore_axis_name="s",
        num_cores=NC, num_subcores=NS)

    @functools.partial(
        pl.kernel,
        mesh=mesh,
        out_type=jax.ShapeDtypeStruct((L * LSTRIDE,), jnp.float32),
        scratch_types=[
            pltpu.VMEM((L, BB), jnp.int32),
            pltpu.VMEM((4, BB, EMB), jnp.float32),
            pltpu.VMEM((2, BLK), jnp.float32),
            pltpu.SemaphoreType.DMA,
            pltpu.SemaphoreType.DMA,
        ],
        compiler_params=pltpu.CompilerParams(
            use_tc_tiling_on_sc=False, needs_layout_passes=False),
    )
    def k(idx_hbm, table_hbm, out_hbm, idx_v, rows_v, tile_v, gsem, wsem):
        wid = lax.axis_index("s") * NC + lax.axis_index("c")
        b0 = wid * BB
        pltpu.sync_copy(idx_hbm.at[:, pl.ds(b0, BB)], idx_v)

        iota = lax.broadcasted_iota(jnp.int32, (16,), 0)
        dst_base = [iota * BB, (iota + 16) * BB]

        def fire_gather(l, p):
            pltpu.async_copy(
                table_hbm.at[idx_v.at[l]], rows_v.at[p], gsem)

        def wait_gather(p):
            pltpu.make_async_copy(
                table_hbm.at[idx_v.at[0]], rows_v.at[p], gsem).wait()

        def transpose(p, q):
            for mg in range(0, NLOADS, 8):
                vals = [
                    rows_v[p, (mg + i) // 2, pl.ds(16 * ((mg + i) % 2), 16)]
                    for i in range(8)
                ]
                for i in range(8):
                    m = mg + i
                    plsc.store_scatter(
                        tile_v.at[q], [dst_base[m % 2] + (m // 2)], vals[i])

        def fire_write(l, q):
            base = l * LSTRIDE + wid * TILE_ELS
            for ct in range(CT):
                pltpu.async_copy(
                    tile_v.at[q].at[pl.ds(ct * TILE_ELS, TILE_ELS)],
                    out_hbm.at[pl.ds(base + ct * NW * TILE_ELS, TILE_ELS)],
                    wsem)

        def wait_write(q):
            for ct in range(CT):
                pltpu.make_async_copy(
                    tile_v.at[q].at[pl.ds(ct * TILE_ELS, TILE_ELS)],
                    out_hbm.at[pl.ds(ct * TILE_ELS, TILE_ELS)],
                    wsem).wait()

        def step(l, p, q):
            @pl.when(l + 3 < L)
            def _():
                fire_gather(l + 3, (p + 3) % 4)

            wait_gather(p)

            @pl.when(l >= 2)
            def _():
                wait_write(q)

            transpose(p, q)
            fire_write(l, q)

        fire_gather(0, 0)
        fire_gather(1, 1)
        fire_gather(2, 2)

        def body(u, carry):
            for r in range(4):
                step(4 * u + r, r, r % 2)
            return carry

        lax.fori_loop(0, L // 4, body, 0)
        wait_write(0)
        wait_write(1)

    return k(idx_lb, table)


def kernel(inputs, table):
    idx_lb = jnp.swapaxes(inputs, 0, 1).astype(jnp.int32)
    flat = _sc_gather(idx_lb, table)
    out5 = flat.reshape(L, CT, NW, 8, BB)
    return jnp.transpose(out5, (2, 4, 0, 1, 3)).reshape(B, L, EMB)
